# baseline (speedup 1.0000x reference)
import functools

import jax
import jax.numpy as jnp
from jax import lax
from jax.experimental import pallas as pl
from jax.experimental.pallas import tpu as pltpu
from jax.experimental.pallas import tpu_sc as plsc

N_NODES = 10000
D = 256
HEADS = 8
HALF = 128
CH = 32
LANES = 16
N_TILES = 16
N_CORES = 2
B_E = 112
N_PAD = 10240
N_G = 10016
ROWS_PER_TILE = N_PAD // N_TILES


_BLK = 1000


def _lin_body(x_ref, w_ref, b_ref, o_ref):
    o_ref[...] = (
        jnp.dot(x_ref[...], w_ref[...], preferred_element_type=jnp.float32)
        + b_ref[...]
    )


def _linear(x, w, b):
    n, d_in = x.shape
    d_out = w.shape[1]
    return pl.pallas_call(
        _lin_body,
        grid=(n // _BLK,),
        in_specs=[
            pl.BlockSpec((_BLK, d_in), lambda i: (i, 0)),
            pl.BlockSpec((d_in, d_out), lambda i: (0, 0)),
            pl.BlockSpec((1, d_out), lambda i: (0, 0)),
        ],
        out_specs=pl.BlockSpec((_BLK, d_out), lambda i: (i, 0)),
        out_shape=jax.ShapeDtypeStruct((n, d_out), jnp.float32),
    )(x, w, b.reshape(1, d_out))


def _proj_body(x_ref, w_ref, wa_ref, p_ref, a_ref):
    x = x_ref[...]
    proj = jnp.dot(x, w_ref[...], preferred_element_type=jnp.float32)
    p_ref[0] = proj[:, :HALF]
    p_ref[1] = proj[:, HALF:]
    a_ref[...] = jnp.dot(x, wa_ref[...], preferred_element_type=jnp.float32)


def _project(x, w, wa):
    n = x.shape[0]
    return pl.pallas_call(
        _proj_body,
        grid=(n // _BLK,),
        in_specs=[
            pl.BlockSpec((_BLK, D), lambda i: (i, 0)),
            pl.BlockSpec((D, D), lambda i: (0, 0)),
            pl.BlockSpec((D, 2 * LANES), lambda i: (0, 0)),
        ],
        out_specs=[
            pl.BlockSpec((2, _BLK, HALF), lambda i: (0, i, 0)),
            pl.BlockSpec((_BLK, 2 * LANES), lambda i: (i, 0)),
        ],
        out_shape=[
            jax.ShapeDtypeStruct((2, n, HALF), jnp.float32),
            jax.ShapeDtypeStruct((n, 2 * LANES), jnp.float32),
        ],
    )(x, w, wa)


def _thin_body(x_ref, wa_ref, a_ref):
    a_ref[...] = jnp.dot(x_ref[...], wa_ref[...], preferred_element_type=jnp.float32)


def _thin(x, wa):
    n = x.shape[0]
    return pl.pallas_call(
        _thin_body,
        grid=(n // _BLK,),
        in_specs=[
            pl.BlockSpec((_BLK, D), lambda i: (i, 0)),
            pl.BlockSpec((D, LANES), lambda i: (0, 0)),
        ],
        out_specs=pl.BlockSpec((_BLK, LANES), lambda i: (i, 0)),
        out_shape=jax.ShapeDtypeStruct((n, LANES), jnp.float32),
    )(x, wa)


def _post_body(*refs, n_gat, relu):
    o_ref = refs[-1]
    res_ref = refs[-2]
    acc = None
    for g in range(n_gat):
        h0, h1, dn, b = refs[4 * g:4 * g + 4]
        dnr = jnp.repeat(dn[...][:, :HEADS] + 1e-16, CH, axis=1)
        t = jnp.concatenate([h0[...], h1[...]], axis=1) / dnr + b[...]
        acc = t if acc is None else acc + t
    acc = acc + res_ref[...]
    if relu:
        acc = jnp.maximum(acc, 0.0)
    o_ref[...] = acc


def _post(gats, res, relu):
    n = res.shape[0]
    args, specs = [], []
    for out2, dn, b in gats:
        args += [out2[:n], out2[N_PAD:N_PAD + n], dn[:n], b.reshape(1, D)]
        specs += [
            pl.BlockSpec((_BLK, HALF), lambda i: (i, 0)),
            pl.BlockSpec((_BLK, HALF), lambda i: (i, 0)),
            pl.BlockSpec((_BLK, LANES), lambda i: (i, 0)),
            pl.BlockSpec((1, D), lambda i: (0, 0)),
        ]
    args.append(res)
    specs.append(pl.BlockSpec((_BLK, D), lambda i: (i, 0)))
    return pl.pallas_call(
        functools.partial(_post_body, n_gat=len(gats), relu=relu),
        grid=(n // _BLK,),
        in_specs=specs,
        out_specs=pl.BlockSpec((_BLK, D), lambda i: (i, 0)),
        out_shape=jax.ShapeDtypeStruct((n, D), jnp.float32),
    )(*args)


_GATHER_DNUMS = lax.GatherDimensionNumbers(
    offset_dims=(), collapsed_slice_dims=(0,), start_index_map=(0,))


def _lane_bcast(vec, idx):
    return lax.gather(vec, idx[:, None], dimension_numbers=_GATHER_DNUMS,
                      slice_sizes=(1,),
                      mode=lax.GatherScatterMode.PROMISE_IN_BOUNDS)


@functools.lru_cache(None)
def _make_edge_kernel(epad):
    per_tile = epad // N_TILES
    nchunks = per_tile // B_E
    assert per_tile % B_E == 0 and nchunks % 2 == 0

    mesh = plsc.VectorSubcoreMesh(core_axis_name="c", subcore_axis_name="s")

    buf_types = []
    for _ in range(2):
        buf_types += [
            pltpu.VMEM((B_E,), jnp.int32),
            pltpu.VMEM((B_E,), jnp.int32),
            pltpu.VMEM((B_E,), jnp.int32),
            pltpu.VMEM((B_E, LANES), jnp.float32),
            pltpu.VMEM((B_E, LANES), jnp.float32),
            pltpu.VMEM((B_E, HALF), jnp.float32),
            pltpu.SemaphoreType.DMA,
            pltpu.SemaphoreType.DMA,
            pltpu.SemaphoreType.DMA,
        ]

    @functools.partial(
        pl.kernel,
        out_type=[
            jax.ShapeDtypeStruct((N_CORES * N_PAD, HALF), jnp.float32),
            jax.ShapeDtypeStruct((N_PAD, LANES), jnp.float32),
        ],
        mesh=mesh,
        compiler_params=pltpu.CompilerParams(use_tc_tiling_on_sc=False),
        scratch_types=buf_types + [
            pltpu.VMEM((B_E, LANES), jnp.float32),
            pltpu.VMEM_SHARED((N_PAD, HALF), jnp.float32),
            pltpu.VMEM_SHARED((N_PAD, LANES), jnp.float32),
        ],
    )
    def edge_kernel(hs2n, a_src, a_dst, srcp, dstp, z128, z16, out2, den,
                    *scratch):
        bufs = (scratch[0:9], scratch[9:18])
        wbuf, accs, dens = scratch[18:]
        c = lax.axis_index("c")
        s = lax.axis_index("s")
        r0 = s * ROWS_PER_TILE

        pltpu.sync_copy(z128.at[pl.ds(r0, ROWS_PER_TILE)],
                        accs.at[pl.ds(r0, ROWS_PER_TILE)])

        @pl.when(c == 0)
        def _():
            pltpu.sync_copy(z16.at[pl.ds(r0, ROWS_PER_TILE)],
                            dens.at[pl.ds(r0, ROWS_PER_TILE)])

        plsc.subcore_barrier()

        tile_base = s * per_tile
        goff = c * N_G
        ooff = c * N_PAD
        col0 = c * (HEADS // N_CORES)

        def fire(k, buf):
            srcb, dstb, gidx, asr, adr, msg, sem_a, sem_b, sem_m = buf
            base = tile_base + k * B_E
            pltpu.sync_copy(srcp.at[pl.ds(base, B_E)], srcb)
            pltpu.sync_copy(dstp.at[pl.ds(base, B_E)], dstb)
            pltpu.async_copy(a_src.at[srcb], asr, sem_a)
            pltpu.async_copy(a_dst.at[dstb], adr, sem_b)

            @plsc.parallel_loop(0, B_E // LANES, unroll=4)
            def gi_body(i):
                sl = pl.ds(i * LANES, LANES)
                gidx[sl] = srcb[sl] + goff

            pltpu.async_copy(hs2n.at[gidx], msg, sem_m)

        def consume(buf):
            srcb, dstb, gidx, asr, adr, msg, sem_a, sem_b, sem_m = buf
            pltpu.make_async_copy(a_src.at[srcb], asr, sem_a).wait()
            pltpu.make_async_copy(a_dst.at[dstb], adr, sem_b).wait()
            pltpu.make_async_copy(hs2n.at[gidx], msg, sem_m).wait()

            @plsc.parallel_loop(0, B_E, unroll=8)
            def ew_body(e):
                row = asr[e, :] + adr[e, :]
                row = jnp.maximum(row, row * 0.2)
                row = jnp.exp(row)
                wbuf[e, :] = row
                for h in range(HEADS // N_CORES):
                    hidx = jnp.full((LANES,), col0 + h, jnp.int32)
                    wv = _lane_bcast(row, hidx)
                    for j in range(CH // LANES):
                        sl = pl.ds(h * CH + j * LANES, LANES)
                        msg[e, sl] = msg[e, sl] * wv

            @pl.when(c == 0)
            def _():
                pltpu.sync_copy(wbuf, dens.at[dstb], add=True)

            pltpu.sync_copy(msg, accs.at[dstb], add=True)

        fire(0, bufs[0])

        def pair_body(p, carry):
            k0 = 2 * p
            fire(k0 + 1, bufs[1])
            consume(bufs[0])

            @pl.when(k0 + 2 < nchunks)
            def _():
                fire(k0 + 2, bufs[0])

            consume(bufs[1])
            return carry

        lax.fori_loop(0, nchunks // 2, pair_body, 0)
        plsc.subcore_barrier()

        pltpu.sync_copy(accs.at[pl.ds(r0, ROWS_PER_TILE)],
                        out2.at[pl.ds(ooff + r0, ROWS_PER_TILE)])

        @pl.when(c == 0)
        def _():
            pltpu.sync_copy(dens.at[pl.ds(r0, ROWS_PER_TILE)],
                            den.at[pl.ds(r0, ROWS_PER_TILE)])

    return edge_kernel


def _edge_phase(proj_src2, a_src, a_dst, src, dst, zeros128, zeros16):
    e = src.shape[0]
    gran = 2 * N_TILES * B_E
    epad = ((e + gran - 1) // gran) * gran
    if epad != e:
        pad_idx = N_NODES + (jnp.arange(epad - e, dtype=src.dtype) % LANES)
        src = jnp.concatenate([src, pad_idx])
        dst = jnp.concatenate([dst, pad_idx])
    k = _make_edge_kernel(epad)
    a_src = jnp.pad(a_src, ((0, N_G - N_NODES), (0, 0)), constant_values=-1000.0)
    a_dst = jnp.pad(a_dst, ((0, N_G - N_NODES), (0, 0)), constant_values=-1000.0)
    hs2n = jnp.pad(proj_src2, ((0, 0), (0, N_G - N_NODES), (0, 0))).reshape(
        N_CORES * N_G, HALF)
    return k(hs2n, a_src, a_dst, src, dst, zeros128, zeros16)


def _att_mats(p):
    w = p["W"].reshape(D, HEADS, CH)
    wa_src = jnp.einsum("dhc,hc->dh", w, p["att_src"])
    wa_dst = jnp.einsum("dhc,hc->dh", w, p["att_dst"])
    pad = jnp.zeros((D, HEADS), jnp.float32)
    return (jnp.concatenate([wa_src, pad, wa_dst, pad], axis=1),
            jnp.concatenate([wa_dst, pad], axis=1))


def kernel(species_x, location_x, params, ei_observes, ei_observed_at, ei_nearby):
    z128 = jnp.zeros((N_PAD, HALF), jnp.float32)
    z16 = jnp.zeros((N_PAD, LANES), jnp.float32)

    ar = jnp.arange(N_NODES, dtype=ei_nearby.dtype)
    ll_src = jnp.concatenate([ei_nearby[0], ar])
    ll_dst = jnp.concatenate([ei_nearby[1], ar])

    hs0 = _linear(species_x, params["Ws"], params["bs"])
    hl0 = _linear(location_x, params["Wl"], params["bl"])

    hs, hl = hs0, hl0
    for layer, final in ((1, False), (2, True)):
        pfx = "c%d_" % layer
        p_ols = params[pfx + "ols"]
        p_slo = params[pfx + "slo"]
        p_ll = params[pfx + "ll"]

        wa_ols, wad_ols = _att_mats(p_ols)
        wa_slo, wad_slo = _att_mats(p_slo)
        wa_ll, _ = _att_mats(p_ll)

        proj_ols, a_ols = _project(hl, p_ols["W"], wa_ols)
        proj_slo, a_slo = _project(hs, p_slo["W"], wa_slo)
        proj_ll, a_ll = _project(hl, p_ll["W"], wa_ll)
        adst_ols = _thin(hs, wad_ols)
        adst_slo = _thin(hl, wad_slo)

        acc_ols, den_ols = _edge_phase(
            proj_ols, a_ols[:, :LANES], adst_ols,
            ei_observes[0], ei_observes[1], z128, z16)
        acc_slo, den_slo = _edge_phase(
            proj_slo, a_slo[:, :LANES], adst_slo,
            ei_observed_at[0], ei_observed_at[1], z128, z16)
        acc_ll, den_ll = _edge_phase(
            proj_ll, a_ll[:, :LANES], a_ll[:, LANES:], ll_src, ll_dst,
            z128, z16)

        hs = _post([(acc_ols, den_ols, p_ols["bias"])], hs, relu=not final)
        hl = _post([(acc_slo, den_slo, p_slo["bias"]),
                    (acc_ll, den_ll, p_ll["bias"])], hl, relu=not final)

    return (hs, hl)


# speedup vs baseline: 1.1838x; 1.1838x over previous
import functools

import jax
import jax.numpy as jnp
from jax import lax
from jax.experimental import pallas as pl
from jax.experimental.pallas import tpu as pltpu
from jax.experimental.pallas import tpu_sc as plsc

N_NODES = 10000
D = 256
HEADS = 8
HALF = 128
CH = 32
LANES = 16
N_TILES = 16
N_CORES = 2
B_E = 112
N_PAD = 10240
N_G = 10016
ROWS_PER_TILE = N_PAD // N_TILES


_BLK = 1000


def _lin_body(x_ref, w_ref, b_ref, o_ref):
    o_ref[...] = (
        jnp.dot(x_ref[...], w_ref[...], preferred_element_type=jnp.float32)
        + b_ref[...]
    )


def _linear(x, w, b):
    n, d_in = x.shape
    d_out = w.shape[1]
    return pl.pallas_call(
        _lin_body,
        grid=(n // _BLK,),
        in_specs=[
            pl.BlockSpec((_BLK, d_in), lambda i: (i, 0)),
            pl.BlockSpec((d_in, d_out), lambda i: (0, 0)),
            pl.BlockSpec((1, d_out), lambda i: (0, 0)),
        ],
        out_specs=pl.BlockSpec((_BLK, d_out), lambda i: (i, 0)),
        out_shape=jax.ShapeDtypeStruct((n, d_out), jnp.float32),
    )(x, w, b.reshape(1, d_out))


def _proj_body(x_ref, w_ref, wa_ref, p_ref, a_ref):
    x = x_ref[...]
    proj = jnp.dot(x, w_ref[...], preferred_element_type=jnp.float32)
    p_ref[0] = proj[:, :HALF]
    p_ref[1] = proj[:, HALF:]
    a_ref[...] = jnp.dot(x, wa_ref[...], preferred_element_type=jnp.float32)


def _project(x, w, wa):
    n = x.shape[0]
    return pl.pallas_call(
        _proj_body,
        grid=(n // _BLK,),
        in_specs=[
            pl.BlockSpec((_BLK, D), lambda i: (i, 0)),
            pl.BlockSpec((D, D), lambda i: (0, 0)),
            pl.BlockSpec((D, 2 * LANES), lambda i: (0, 0)),
        ],
        out_specs=[
            pl.BlockSpec((2, _BLK, HALF), lambda i: (0, i, 0)),
            pl.BlockSpec((_BLK, 2 * LANES), lambda i: (i, 0)),
        ],
        out_shape=[
            jax.ShapeDtypeStruct((2, n, HALF), jnp.float32),
            jax.ShapeDtypeStruct((n, 2 * LANES), jnp.float32),
        ],
    )(x, w, wa)


def _thin_body(x_ref, wa_ref, a_ref):
    a_ref[...] = jnp.dot(x_ref[...], wa_ref[...], preferred_element_type=jnp.float32)


def _thin(x, wa):
    n = x.shape[0]
    return pl.pallas_call(
        _thin_body,
        grid=(n // _BLK,),
        in_specs=[
            pl.BlockSpec((_BLK, D), lambda i: (i, 0)),
            pl.BlockSpec((D, LANES), lambda i: (0, 0)),
        ],
        out_specs=pl.BlockSpec((_BLK, LANES), lambda i: (i, 0)),
        out_shape=jax.ShapeDtypeStruct((n, LANES), jnp.float32),
    )(x, wa)


def _post_body(*refs, n_gat, relu):
    o_ref = refs[-1]
    res_ref = refs[-2]
    acc = None
    for g in range(n_gat):
        h0, h1, dn, b = refs[4 * g:4 * g + 4]
        dnr = jnp.repeat(dn[...][:, :HEADS] + 1e-16, CH, axis=1)
        t = jnp.concatenate([h0[...], h1[...]], axis=1) / dnr + b[...]
        acc = t if acc is None else acc + t
    acc = acc + res_ref[...]
    if relu:
        acc = jnp.maximum(acc, 0.0)
    o_ref[...] = acc


def _post(gats, res, relu):
    n = res.shape[0]
    args, specs = [], []
    for out2, dn, b in gats:
        args += [out2[:n], out2[N_PAD:N_PAD + n], dn[:n], b.reshape(1, D)]
        specs += [
            pl.BlockSpec((_BLK, HALF), lambda i: (i, 0)),
            pl.BlockSpec((_BLK, HALF), lambda i: (i, 0)),
            pl.BlockSpec((_BLK, LANES), lambda i: (i, 0)),
            pl.BlockSpec((1, D), lambda i: (0, 0)),
        ]
    args.append(res)
    specs.append(pl.BlockSpec((_BLK, D), lambda i: (i, 0)))
    return pl.pallas_call(
        functools.partial(_post_body, n_gat=len(gats), relu=relu),
        grid=(n // _BLK,),
        in_specs=specs,
        out_specs=pl.BlockSpec((_BLK, D), lambda i: (i, 0)),
        out_shape=jax.ShapeDtypeStruct((n, D), jnp.float32),
    )(*args)


_GATHER_DNUMS = lax.GatherDimensionNumbers(
    offset_dims=(), collapsed_slice_dims=(0,), start_index_map=(0,))


def _lane_bcast(vec, idx):
    return lax.gather(vec, idx[:, None], dimension_numbers=_GATHER_DNUMS,
                      slice_sizes=(1,),
                      mode=lax.GatherScatterMode.PROMISE_IN_BOUNDS)


@functools.lru_cache(None)
def _make_edge_kernel(epad):
    per_tile = epad // N_TILES
    nchunks = per_tile // B_E
    assert per_tile % B_E == 0 and nchunks % 2 == 0

    mesh = plsc.VectorSubcoreMesh(core_axis_name="c", subcore_axis_name="s")

    buf_types = []
    for _ in range(2):
        buf_types += [
            pltpu.VMEM((B_E,), jnp.int32),
            pltpu.VMEM((B_E,), jnp.int32),
            pltpu.VMEM((B_E,), jnp.int32),
            pltpu.VMEM((B_E, LANES), jnp.float32),
            pltpu.VMEM((B_E, LANES), jnp.float32),
            pltpu.VMEM((B_E, HALF), jnp.float32),
            pltpu.SemaphoreType.DMA,
            pltpu.SemaphoreType.DMA,
            pltpu.SemaphoreType.DMA,
        ]

    @functools.partial(
        pl.kernel,
        out_type=[
            jax.ShapeDtypeStruct((N_CORES * N_PAD, HALF), jnp.float32),
            jax.ShapeDtypeStruct((N_PAD, LANES), jnp.float32),
        ],
        mesh=mesh,
        compiler_params=pltpu.CompilerParams(use_tc_tiling_on_sc=False),
        scratch_types=buf_types + [
            pltpu.VMEM((B_E, LANES), jnp.float32),
            pltpu.VMEM_SHARED((N_PAD, HALF), jnp.float32),
            pltpu.VMEM_SHARED((N_PAD, LANES), jnp.float32),
        ],
    )
    def edge_kernel(hs2n, a_src, a_dst, srcp, dstp, z128, z16, out2, den,
                    *scratch):
        bufs = (scratch[0:9], scratch[9:18])
        wbuf, accs, dens = scratch[18:]
        c = lax.axis_index("c")
        s = lax.axis_index("s")
        r0 = s * ROWS_PER_TILE

        pltpu.sync_copy(z128.at[pl.ds(r0, ROWS_PER_TILE)],
                        accs.at[pl.ds(r0, ROWS_PER_TILE)])

        @pl.when(c == 0)
        def _():
            pltpu.sync_copy(z16.at[pl.ds(r0, ROWS_PER_TILE)],
                            dens.at[pl.ds(r0, ROWS_PER_TILE)])

        plsc.subcore_barrier()

        tile_base = s * per_tile
        goff = c * N_G
        ooff = c * N_PAD
        col0 = c * (HEADS // N_CORES)

        def fire(k, buf):
            srcb, dstb, gidx, asr, adr, msg, sem_a, sem_b, sem_m = buf
            base = tile_base + k * B_E
            cp_s = pltpu.async_copy(srcp.at[pl.ds(base, B_E)], srcb, sem_a)
            cp_d = pltpu.async_copy(dstp.at[pl.ds(base, B_E)], dstb, sem_b)
            cp_s.wait()
            cp_d.wait()
            pltpu.async_copy(a_src.at[srcb], asr, sem_a)
            pltpu.async_copy(a_dst.at[dstb], adr, sem_b)

            @plsc.parallel_loop(0, B_E // LANES, unroll=4)
            def gi_body(i):
                sl = pl.ds(i * LANES, LANES)
                gidx[sl] = srcb[sl] + goff

            pltpu.async_copy(hs2n.at[gidx], msg, sem_m)

        def consume(buf):
            srcb, dstb, gidx, asr, adr, msg, sem_a, sem_b, sem_m = buf
            pltpu.make_async_copy(a_src.at[srcb], asr, sem_a).wait()
            pltpu.make_async_copy(a_dst.at[dstb], adr, sem_b).wait()
            pltpu.make_async_copy(hs2n.at[gidx], msg, sem_m).wait()

            @plsc.parallel_loop(0, B_E, unroll=4)
            def ew_body(e):
                row = asr[e, :] + adr[e, :]
                row = jnp.maximum(row, row * 0.2)
                row = jnp.exp(row)
                wbuf[e, :] = row
                for h in range(HEADS // N_CORES):
                    hidx = jnp.full((LANES,), col0 + h, jnp.int32)
                    wv = _lane_bcast(row, hidx)
                    for j in range(CH // LANES):
                        sl = pl.ds(h * CH + j * LANES, LANES)
                        msg[e, sl] = msg[e, sl] * wv

            @pl.when(c == 0)
            def _():
                pltpu.sync_copy(wbuf, dens.at[dstb], add=True)

            pltpu.sync_copy(msg, accs.at[dstb], add=True)

        fire(0, bufs[0])

        def pair_body(p, carry):
            k0 = 2 * p
            fire(k0 + 1, bufs[1])
            consume(bufs[0])

            @pl.when(k0 + 2 < nchunks)
            def _():
                fire(k0 + 2, bufs[0])

            consume(bufs[1])
            return carry

        lax.fori_loop(0, nchunks // 2, pair_body, 0)
        plsc.subcore_barrier()

        pltpu.sync_copy(accs.at[pl.ds(r0, ROWS_PER_TILE)],
                        out2.at[pl.ds(ooff + r0, ROWS_PER_TILE)])

        @pl.when(c == 0)
        def _():
            pltpu.sync_copy(dens.at[pl.ds(r0, ROWS_PER_TILE)],
                            den.at[pl.ds(r0, ROWS_PER_TILE)])

    return edge_kernel


def _edge_phase(proj_src2, a_src, a_dst, src, dst, zeros128, zeros16):
    e = src.shape[0]
    gran = 2 * N_TILES * B_E
    epad = ((e + gran - 1) // gran) * gran
    if epad != e:
        pad_idx = N_NODES + (jnp.arange(epad - e, dtype=src.dtype) % LANES)
        src = jnp.concatenate([src, pad_idx])
        dst = jnp.concatenate([dst, pad_idx])
    k = _make_edge_kernel(epad)
    a_src = jnp.pad(a_src, ((0, N_G - N_NODES), (0, 0)), constant_values=-1000.0)
    a_dst = jnp.pad(a_dst, ((0, N_G - N_NODES), (0, 0)), constant_values=-1000.0)
    hs2n = jnp.pad(proj_src2, ((0, 0), (0, N_G - N_NODES), (0, 0))).reshape(
        N_CORES * N_G, HALF)
    return k(hs2n, a_src, a_dst, src, dst, zeros128, zeros16)


def _att_mats(p):
    w = p["W"].reshape(D, HEADS, CH)
    wa_src = jnp.einsum("dhc,hc->dh", w, p["att_src"])
    wa_dst = jnp.einsum("dhc,hc->dh", w, p["att_dst"])
    pad = jnp.zeros((D, HEADS), jnp.float32)
    return (jnp.concatenate([wa_src, pad, wa_dst, pad], axis=1),
            jnp.concatenate([wa_dst, pad], axis=1))


def kernel(species_x, location_x, params, ei_observes, ei_observed_at, ei_nearby):
    z128 = jnp.zeros((N_PAD, HALF), jnp.float32)
    z16 = jnp.zeros((N_PAD, LANES), jnp.float32)

    ar = jnp.arange(N_NODES, dtype=ei_nearby.dtype)
    ll_src = jnp.concatenate([ei_nearby[0], ar])
    ll_dst = jnp.concatenate([ei_nearby[1], ar])

    hs0 = _linear(species_x, params["Ws"], params["bs"])
    hl0 = _linear(location_x, params["Wl"], params["bl"])

    hs, hl = hs0, hl0
    for layer, final in ((1, False), (2, True)):
        pfx = "c%d_" % layer
        p_ols = params[pfx + "ols"]
        p_slo = params[pfx + "slo"]
        p_ll = params[pfx + "ll"]

        wa_ols, wad_ols = _att_mats(p_ols)
        wa_slo, wad_slo = _att_mats(p_slo)
        wa_ll, _ = _att_mats(p_ll)

        proj_ols, a_ols = _project(hl, p_ols["W"], wa_ols)
        proj_slo, a_slo = _project(hs, p_slo["W"], wa_slo)
        proj_ll, a_ll = _project(hl, p_ll["W"], wa_ll)
        adst_ols = _thin(hs, wad_ols)
        adst_slo = _thin(hl, wad_slo)

        acc_ols, den_ols = _edge_phase(
            proj_ols, a_ols[:, :LANES], adst_ols,
            ei_observes[0], ei_observes[1], z128, z16)
        acc_slo, den_slo = _edge_phase(
            proj_slo, a_slo[:, :LANES], adst_slo,
            ei_observed_at[0], ei_observed_at[1], z128, z16)
        acc_ll, den_ll = _edge_phase(
            proj_ll, a_ll[:, :LANES], a_ll[:, LANES:], ll_src, ll_dst,
            z128, z16)

        hs = _post([(acc_ols, den_ols, p_ols["bias"])], hs, relu=not final)
        hl = _post([(acc_slo, den_slo, p_slo["bias"]),
                    (acc_ll, den_ll, p_ll["bias"])], hl, relu=not final)

    return (hs, hl)
